# baseline (speedup 1.0000x reference)
import functools

import jax
import jax.numpy as jnp
from jax.experimental import pallas as pl
from jax.experimental.pallas import tpu as pltpu

_MARGIN = 1.0
_LANES = 128
_N = 8192
_NI = _N // _LANES
_NJ = _N // _LANES


def _pair_kernel(pi_ref, ti_ref, prow_ref, trow_ref, out_t_ref, out_c_ref):
    pcol = jnp.transpose(pi_ref[0], (1, 0))
    tcol = jnp.transpose(ti_ref[0], (1, 0))
    acol = pcol + _MARGIN

    def body(j, carry):
        acc_t, acc_c = carry
        js = pl.multiple_of(j * _LANES, _LANES)
        prow = prow_ref[:, pl.ds(js, _LANES)]
        trow = trow_ref[:, pl.ds(js, _LANES)]
        mask = tcol < trow
        term = jnp.where(mask, jnp.maximum(acol - prow, 0.0), 0.0)
        cnt = jnp.where(mask, 1.0, 0.0)
        term = term.reshape(16, 8, _LANES)
        cnt = cnt.reshape(16, 8, _LANES)
        pt = term[0]
        pc = cnt[0]
        for k in range(1, 16):
            pt = pt + term[k]
            pc = pc + cnt[k]
        return acc_t + pt, acc_c + pc

    zero = jnp.zeros((8, _LANES), jnp.float32)
    acc_t, acc_c = jax.lax.fori_loop(0, _NJ, body, (zero, zero))
    out_t_ref[0] = acc_t
    out_c_ref[0] = acc_c


def _finalize_kernel(pt_ref, pc_ref, out_ref):
    total = jnp.sum(pt_ref[...])
    count = jnp.sum(pc_ref[...])
    out_ref[0, 0] = jnp.where(count > 0.0, total / count, 0.0)


@jax.jit
def kernel(pred, target):
    p = pred.reshape(-1).astype(jnp.float32)
    t = target.reshape(-1).astype(jnp.float32)
    p_row = p.reshape(1, _N)
    t_row = t.reshape(1, _N)
    p_i3 = p.reshape(_NI, 1, _LANES)
    t_i3 = t.reshape(_NI, 1, _LANES)

    part_t, part_c = pl.pallas_call(
        _pair_kernel,
        grid=(_NI,),
        in_specs=[
            pl.BlockSpec((1, 1, _LANES), lambda i: (i, 0, 0)),
            pl.BlockSpec((1, 1, _LANES), lambda i: (i, 0, 0)),
            pl.BlockSpec((1, _N), lambda i: (0, 0)),
            pl.BlockSpec((1, _N), lambda i: (0, 0)),
        ],
        out_specs=[
            pl.BlockSpec((1, 8, _LANES), lambda i: (i, 0, 0)),
            pl.BlockSpec((1, 8, _LANES), lambda i: (i, 0, 0)),
        ],
        out_shape=[
            jax.ShapeDtypeStruct((_NI, 8, _LANES), jnp.float32),
            jax.ShapeDtypeStruct((_NI, 8, _LANES), jnp.float32),
        ],
        compiler_params=pltpu.CompilerParams(
            dimension_semantics=("parallel",),
        ),
    )(p_i3, t_i3, p_row, t_row)

    loss = pl.pallas_call(
        _finalize_kernel,
        out_specs=pl.BlockSpec(memory_space=pltpu.SMEM),
        out_shape=jax.ShapeDtypeStruct((1, 1), jnp.float32),
    )(part_t, part_c)

    return loss[0, 0]


# speedup vs baseline: 4.0395x; 4.0395x over previous
import functools

import jax
import jax.numpy as jnp
from jax.experimental import pallas as pl
from jax.experimental.pallas import tpu as pltpu

_MARGIN = 1.0
_LANES = 128
_N = 8192
_NI = _N // _LANES
_NJ = _N // _LANES


def _pair_kernel(pi_ref, ti_ref, prow_ref, trow_ref, out_t_ref, out_c_ref):
    acol = jnp.transpose(pi_ref[0], (1, 0)) + _MARGIN
    acolb = jnp.broadcast_to(acol, (_LANES, _LANES))
    tcolb = jnp.broadcast_to(
        jnp.transpose(ti_ref[0], (1, 0)), (_LANES, _LANES))

    def body(j, carry):
        acc_t, acc_c = carry
        js = pl.multiple_of(j * _LANES, _LANES)
        prow = prow_ref[:, pl.ds(js, _LANES)]
        trow = trow_ref[:, pl.ds(js, _LANES)]
        mask = tcolb < trow
        term = jnp.where(mask, jnp.maximum(acolb - prow, 0.0), 0.0)
        cnt = jnp.where(mask, 1.0, 0.0)
        term = term.reshape(16, 8, _LANES)
        cnt = cnt.reshape(16, 8, _LANES)
        pt = term[0]
        pc = cnt[0]
        for k in range(1, 16):
            pt = pt + term[k]
            pc = pc + cnt[k]
        return acc_t + pt, acc_c + pc

    zero = jnp.zeros((8, _LANES), jnp.float32)
    acc_t, acc_c = jax.lax.fori_loop(0, _NJ, body, (zero, zero))
    out_t_ref[0] = acc_t
    out_c_ref[0] = acc_c


def _finalize_kernel(pt_ref, pc_ref, out_ref):
    total = jnp.sum(pt_ref[...])
    count = jnp.sum(pc_ref[...])
    out_ref[0, 0] = jnp.where(count > 0.0, total / count, 0.0)


@jax.jit
def kernel(pred, target):
    p = pred.reshape(-1).astype(jnp.float32)
    t = target.reshape(-1).astype(jnp.float32)
    p_row = p.reshape(1, _N)
    t_row = t.reshape(1, _N)
    p_i3 = p.reshape(_NI, 1, _LANES)
    t_i3 = t.reshape(_NI, 1, _LANES)

    part_t, part_c = pl.pallas_call(
        _pair_kernel,
        grid=(_NI,),
        in_specs=[
            pl.BlockSpec((1, 1, _LANES), lambda i: (i, 0, 0)),
            pl.BlockSpec((1, 1, _LANES), lambda i: (i, 0, 0)),
            pl.BlockSpec((1, _N), lambda i: (0, 0)),
            pl.BlockSpec((1, _N), lambda i: (0, 0)),
        ],
        out_specs=[
            pl.BlockSpec((1, 8, _LANES), lambda i: (i, 0, 0)),
            pl.BlockSpec((1, 8, _LANES), lambda i: (i, 0, 0)),
        ],
        out_shape=[
            jax.ShapeDtypeStruct((_NI, 8, _LANES), jnp.float32),
            jax.ShapeDtypeStruct((_NI, 8, _LANES), jnp.float32),
        ],
        compiler_params=pltpu.CompilerParams(
            dimension_semantics=("parallel",),
        ),
    )(p_i3, t_i3, p_row, t_row)

    loss = pl.pallas_call(
        _finalize_kernel,
        out_specs=pl.BlockSpec(memory_space=pltpu.SMEM),
        out_shape=jax.ShapeDtypeStruct((1, 1), jnp.float32),
    )(part_t, part_c)

    return loss[0, 0]


# speedup vs baseline: 5.5191x; 1.3663x over previous
import functools

import jax
import jax.numpy as jnp
from jax.experimental import pallas as pl
from jax.experimental.pallas import tpu as pltpu

_MARGIN = 1.0
_LANES = 128
_N = 8192
_NI = _N // _LANES
_NJ = _N // _LANES
_NCLS = 20


def _pair_kernel(pi_ref, ti_ref, prow_ref, trow_ref, out_t_ref):
    acol = jnp.transpose(pi_ref[0], (1, 0)) + _MARGIN
    acolb = jnp.broadcast_to(acol, (_LANES, _LANES))
    tcolb = jnp.broadcast_to(
        jnp.transpose(ti_ref[0], (1, 0)), (_LANES, _LANES))

    def body(j, acc):
        js = pl.multiple_of(j * _LANES, _LANES)
        prow = prow_ref[:, pl.ds(js, _LANES)]
        trow = trow_ref[:, pl.ds(js, _LANES)]
        mask = tcolb < trow
        term = jnp.where(mask, jnp.maximum(acolb - prow, 0.0), 0.0)
        return acc + term

    acc = jax.lax.fori_loop(
        0, _NJ, body, jnp.zeros((_LANES, _LANES), jnp.float32))

    acc = acc.reshape(16, 8, _LANES)
    parts = [acc[k] for k in range(16)]
    while len(parts) > 1:
        parts = [a + b for a, b in zip(parts[::2], parts[1::2])]
    out_t_ref[0] = parts[0]


def _finalize_kernel(pt_ref, ti_ref, out_ref):
    total = jnp.sum(pt_ref[...])
    t = ti_ref[...]
    sum_sq = jnp.int32(0)
    for v in range(_NCLS):
        c = jnp.sum(jnp.where(t == v, 1, 0))
        sum_sq = sum_sq + c * c
    count = (jnp.int32(_N * _N) - sum_sq) // 2
    countf = count.astype(jnp.float32)
    out_ref[0, 0] = jnp.where(count > 0, total / countf, 0.0)


@jax.jit
def kernel(pred, target):
    p = pred.reshape(-1).astype(jnp.float32)
    tf = target.reshape(-1).astype(jnp.float32)
    p_row = p.reshape(1, _N)
    t_row = tf.reshape(1, _N)
    p_i3 = p.reshape(_NI, 1, _LANES)
    t_i3 = tf.reshape(_NI, 1, _LANES)
    t_i32 = target.reshape(1, _N).astype(jnp.int32)

    part_t = pl.pallas_call(
        _pair_kernel,
        grid=(_NI,),
        in_specs=[
            pl.BlockSpec((1, 1, _LANES), lambda i: (i, 0, 0)),
            pl.BlockSpec((1, 1, _LANES), lambda i: (i, 0, 0)),
            pl.BlockSpec((1, _N), lambda i: (0, 0)),
            pl.BlockSpec((1, _N), lambda i: (0, 0)),
        ],
        out_specs=pl.BlockSpec((1, 8, _LANES), lambda i: (i, 0, 0)),
        out_shape=jax.ShapeDtypeStruct((_NI, 8, _LANES), jnp.float32),
        compiler_params=pltpu.CompilerParams(
            dimension_semantics=("parallel",),
        ),
    )(p_i3, t_i3, p_row, t_row)

    loss = pl.pallas_call(
        _finalize_kernel,
        out_specs=pl.BlockSpec(memory_space=pltpu.SMEM),
        out_shape=jax.ShapeDtypeStruct((1, 1), jnp.float32),
    )(part_t, t_i32)

    return loss[0, 0]


# speedup vs baseline: 5.6127x; 1.0170x over previous
import functools

import jax
import jax.numpy as jnp
from jax.experimental import pallas as pl
from jax.experimental.pallas import tpu as pltpu

_MARGIN = 1.0
_LANES = 128
_N = 8192
_NI = _N // _LANES
_NJ = _N // _LANES
_NCORES = 2
_NI_PER_CORE = _NI // _NCORES
_NCLS = 20


def _pair_kernel(prow_ref, trow_ref, out_t_ref):
    pid = pl.program_id(0)

    def ibody(oi, tot):
        ii = pid * _NI_PER_CORE + oi
        istart = pl.multiple_of(ii * _LANES, _LANES)
        pi = prow_ref[:, pl.ds(istart, _LANES)]
        ti = trow_ref[:, pl.ds(istart, _LANES)]
        acol = jnp.transpose(pi, (1, 0)) + _MARGIN
        acolb = jnp.broadcast_to(acol, (_LANES, _LANES))
        tcolb = jnp.broadcast_to(jnp.transpose(ti, (1, 0)), (_LANES, _LANES))

        def jbody(j, acc):
            js = pl.multiple_of(j * _LANES, _LANES)
            prow = prow_ref[:, pl.ds(js, _LANES)]
            trow = trow_ref[:, pl.ds(js, _LANES)]
            mask = tcolb < trow
            term = jnp.where(mask, jnp.maximum(acolb - prow, 0.0), 0.0)
            return acc + term

        acc = jax.lax.fori_loop(
            0, _NJ, jbody, jnp.zeros((_LANES, _LANES), jnp.float32))

        acc = acc.reshape(16, 8, _LANES)
        parts = [acc[k] for k in range(16)]
        while len(parts) > 1:
            parts = [a + b for a, b in zip(parts[::2], parts[1::2])]
        return tot + parts[0]

    out_t_ref[0] = jax.lax.fori_loop(
        0, _NI_PER_CORE, ibody, jnp.zeros((8, _LANES), jnp.float32))


def _finalize_kernel(pt_ref, ti_ref, out_ref):
    total = jnp.sum(pt_ref[...])
    t = ti_ref[...]
    sum_sq = jnp.int32(0)
    for v in range(_NCLS):
        c = jnp.sum(jnp.where(t == v, 1, 0))
        sum_sq = sum_sq + c * c
    count = (jnp.int32(_N * _N) - sum_sq) // 2
    countf = count.astype(jnp.float32)
    out_ref[0, 0] = jnp.where(count > 0, total / countf, 0.0)


@jax.jit
def kernel(pred, target):
    p_row = pred.reshape(1, _N).astype(jnp.float32)
    t_row = target.reshape(1, _N).astype(jnp.float32)
    t_i32 = target.reshape(1, _N).astype(jnp.int32)

    part_t = pl.pallas_call(
        _pair_kernel,
        grid=(_NCORES,),
        in_specs=[
            pl.BlockSpec((1, _N), lambda i: (0, 0)),
            pl.BlockSpec((1, _N), lambda i: (0, 0)),
        ],
        out_specs=pl.BlockSpec((1, 8, _LANES), lambda i: (i, 0, 0)),
        out_shape=jax.ShapeDtypeStruct((_NCORES, 8, _LANES), jnp.float32),
        compiler_params=pltpu.CompilerParams(
            dimension_semantics=("parallel",),
        ),
    )(p_row, t_row)

    loss = pl.pallas_call(
        _finalize_kernel,
        out_specs=pl.BlockSpec(memory_space=pltpu.SMEM),
        out_shape=jax.ShapeDtypeStruct((1, 1), jnp.float32),
    )(part_t, t_i32)

    return loss[0, 0]


# speedup vs baseline: 5.6616x; 1.0087x over previous
import functools

import jax
import jax.numpy as jnp
from jax.experimental import pallas as pl
from jax.experimental.pallas import tpu as pltpu

_MARGIN = 1.0
_LANES = 128
_N = 8192
_NI = _N // _LANES
_NJ = _N // _LANES
_NCLS = 20


def _pair_kernel(prow_ref, trow_ref, out_t_ref, acolb_ref, tcolb_ref):
    def ibody(ii, tot):
        istart = pl.multiple_of(ii * _LANES, _LANES)
        pi = prow_ref[:, pl.ds(istart, _LANES)]
        ti = trow_ref[:, pl.ds(istart, _LANES)]
        acol = jnp.transpose(pi, (1, 0)) + _MARGIN
        acolb_ref[...] = jnp.broadcast_to(acol, (_LANES, _LANES))
        tcolb_ref[...] = jnp.broadcast_to(
            jnp.transpose(ti, (1, 0)), (_LANES, _LANES))

        def jbody(j, acc):
            js = pl.multiple_of(j * _LANES, _LANES)
            prow = prow_ref[:, pl.ds(js, _LANES)]
            trow = trow_ref[:, pl.ds(js, _LANES)]
            mask = tcolb_ref[...] < trow
            term = jnp.where(
                mask, jnp.maximum(acolb_ref[...] - prow, 0.0), 0.0)
            return acc + term

        acc = jax.lax.fori_loop(
            0, _NJ, jbody, jnp.zeros((_LANES, _LANES), jnp.float32),
            unroll=2)

        acc = acc.reshape(16, 8, _LANES)
        parts = [acc[k] for k in range(16)]
        while len(parts) > 1:
            parts = [a + b for a, b in zip(parts[::2], parts[1::2])]
        return tot + parts[0]

    out_t_ref[...] = jax.lax.fori_loop(
        0, _NI, ibody, jnp.zeros((8, _LANES), jnp.float32))


def _finalize_kernel(pt_ref, ti_ref, out_ref):
    total = jnp.sum(pt_ref[...])
    t = ti_ref[...]
    sum_sq = jnp.int32(0)
    for v in range(_NCLS):
        c = jnp.sum(jnp.where(t == v, 1, 0))
        sum_sq = sum_sq + c * c
    count = (jnp.int32(_N * _N) - sum_sq) // 2
    countf = count.astype(jnp.float32)
    out_ref[0, 0] = jnp.where(count > 0, total / countf, 0.0)


@jax.jit
def kernel(pred, target):
    p_row = pred.reshape(1, _N).astype(jnp.float32)
    t_row = target.reshape(1, _N).astype(jnp.float32)
    t_i32 = target.reshape(1, _N).astype(jnp.int32)

    part_t = pl.pallas_call(
        _pair_kernel,
        out_shape=jax.ShapeDtypeStruct((8, _LANES), jnp.float32),
        scratch_shapes=[
            pltpu.VMEM((_LANES, _LANES), jnp.float32),
            pltpu.VMEM((_LANES, _LANES), jnp.float32),
        ],
    )(p_row, t_row)

    loss = pl.pallas_call(
        _finalize_kernel,
        out_specs=pl.BlockSpec(memory_space=pltpu.SMEM),
        out_shape=jax.ShapeDtypeStruct((1, 1), jnp.float32),
    )(part_t, t_i32)

    return loss[0, 0]


# speedup vs baseline: 6.0903x; 1.0757x over previous
import functools

import jax
import jax.numpy as jnp
from jax.experimental import pallas as pl
from jax.experimental.pallas import tpu as pltpu

_MARGIN = 1.0
_BIG = 1024.0
_LANES = 128
_N = 8192
_NI = _N // _LANES
_NJ = _N // _LANES
_NCLS = 20


def _pair_kernel(prow_ref, trow_ref, qrow_ref, out_t_ref, ab_ref, tb_ref):
    def sbody(c, carry):
        cs = pl.multiple_of(c * _LANES, _LANES)
        pi = prow_ref[:, pl.ds(cs, _LANES)]
        ti = trow_ref[:, pl.ds(cs, _LANES)]
        acol = jnp.transpose(pi, (1, 0)) + _MARGIN
        tcol = jnp.transpose(ti, (1, 0)) * _BIG
        ab_ref[pl.ds(cs, _LANES), :] = jnp.broadcast_to(
            acol, (_LANES, _LANES))
        tb_ref[pl.ds(cs, _LANES), :] = jnp.broadcast_to(
            tcol, (_LANES, _LANES))
        return carry

    jax.lax.fori_loop(0, _NI, sbody, 0)

    def ibody(ii, tot):
        ist = pl.multiple_of(ii * 64, 64)
        acolb = ab_ref[pl.ds(ist, 64), :]
        tcolb = tb_ref[pl.ds(ist, 64), :]

        def jbody(j, acc):
            js = pl.multiple_of(j * _LANES, _LANES)
            prow = prow_ref[:, pl.ds(js, _LANES)]
            qrow = qrow_ref[:, pl.ds(js, _LANES)]
            x = acolb - prow
            y = qrow - tcolb
            term = jnp.maximum(jnp.minimum(x, y), 0.0)
            return acc + term

        acc = jax.lax.fori_loop(
            0, _NJ, jbody, jnp.zeros((64, _LANES), jnp.float32),
            unroll=4)

        acc = acc.reshape(8, 8, _LANES)
        parts = [acc[k] for k in range(8)]
        while len(parts) > 1:
            parts = [a + b for a, b in zip(parts[::2], parts[1::2])]
        return tot + parts[0]

    out_t_ref[...] = jax.lax.fori_loop(
        0, 2 * _NI, ibody, jnp.zeros((8, _LANES), jnp.float32))


def _finalize_kernel(pt_ref, ti_ref, out_ref):
    total = jnp.sum(pt_ref[...])
    t = ti_ref[...]
    sum_sq = jnp.int32(0)
    for v in range(_NCLS):
        c = jnp.sum(jnp.where(t == v, 1, 0))
        sum_sq = sum_sq + c * c
    count = (jnp.int32(_N * _N) - sum_sq) // 2
    countf = count.astype(jnp.float32)
    out_ref[0, 0] = jnp.where(count > 0, total / countf, 0.0)


@jax.jit
def kernel(pred, target):
    p_row = pred.reshape(1, _N).astype(jnp.float32)
    t_row = target.reshape(1, _N).astype(jnp.float32)
    q_row = t_row * _BIG - (_BIG / 2)
    t_i32 = target.reshape(1, _N).astype(jnp.int32)

    part_t = pl.pallas_call(
        _pair_kernel,
        out_shape=jax.ShapeDtypeStruct((8, _LANES), jnp.float32),
        scratch_shapes=[
            pltpu.VMEM((_N, _LANES), jnp.float32),
            pltpu.VMEM((_N, _LANES), jnp.float32),
        ],
    )(p_row, t_row, q_row)

    loss = pl.pallas_call(
        _finalize_kernel,
        out_specs=pl.BlockSpec(memory_space=pltpu.SMEM),
        out_shape=jax.ShapeDtypeStruct((1, 1), jnp.float32),
    )(part_t, t_i32)

    return loss[0, 0]


# speedup vs baseline: 6.9734x; 1.1450x over previous
import functools

import jax
import jax.numpy as jnp
from jax.experimental import pallas as pl
from jax.experimental.pallas import tpu as pltpu

_MARGIN = 1.0
_LANES = 128
_N = 8192
_NI = _N // _LANES
_NJ = _N // _LANES
_NCLS = 20


def _pair_kernel(prow_ref, trow_ref, out_t_ref, ab_ref, tb_ref):
    def sbody(c, carry):
        cs = pl.multiple_of(c * _LANES, _LANES)
        pi = prow_ref[:, pl.ds(cs, _LANES)]
        ti = trow_ref[:, pl.ds(cs, _LANES)]
        acol = jnp.transpose(pi, (1, 0)) + _MARGIN
        ab_ref[pl.ds(cs, _LANES), :] = jnp.broadcast_to(
            acol, (_LANES, _LANES))
        tb_ref[pl.ds(cs, _LANES), :] = jnp.broadcast_to(
            jnp.transpose(ti, (1, 0)), (_LANES, _LANES))
        return carry

    jax.lax.fori_loop(0, _NI, sbody, 0)

    def ibody(ii, tot):
        ist = pl.multiple_of(ii * 64, 64)
        acolb = ab_ref[pl.ds(ist, 64), :]
        tcolb = tb_ref[pl.ds(ist, 64), :]

        def jbody(j, acc):
            js = pl.multiple_of(j * _LANES, _LANES)
            prow = prow_ref[:, pl.ds(js, _LANES)]
            trow = trow_ref[:, pl.ds(js, _LANES)]
            mask = tcolb < trow
            term = jnp.where(mask, jnp.maximum(acolb - prow, 0.0), 0.0)
            return acc + term

        acc = jax.lax.fori_loop(
            0, _NJ, jbody, jnp.zeros((64, _LANES), jnp.float32),
            unroll=8)

        acc = acc.reshape(8, 8, _LANES)
        parts = [acc[k] for k in range(8)]
        while len(parts) > 1:
            parts = [a + b for a, b in zip(parts[::2], parts[1::2])]
        return tot + parts[0]

    out_t_ref[...] = jax.lax.fori_loop(
        0, 2 * _NI, ibody, jnp.zeros((8, _LANES), jnp.float32))


def _finalize_kernel(pt_ref, ti_ref, out_ref):
    total = jnp.sum(pt_ref[...])
    t = ti_ref[...]
    sum_sq = jnp.int32(0)
    for v in range(_NCLS):
        c = jnp.sum(jnp.where(t == v, 1, 0))
        sum_sq = sum_sq + c * c
    count = (jnp.int32(_N * _N) - sum_sq) // 2
    countf = count.astype(jnp.float32)
    out_ref[0, 0] = jnp.where(count > 0, total / countf, 0.0)


@jax.jit
def kernel(pred, target):
    p_row = pred.reshape(1, _N).astype(jnp.float32)
    t_row = target.reshape(1, _N).astype(jnp.float32)
    t_i32 = target.reshape(1, _N).astype(jnp.int32)

    part_t = pl.pallas_call(
        _pair_kernel,
        out_shape=jax.ShapeDtypeStruct((8, _LANES), jnp.float32),
        scratch_shapes=[
            pltpu.VMEM((_N, _LANES), jnp.float32),
            pltpu.VMEM((_N, _LANES), jnp.float32),
        ],
    )(p_row, t_row)

    loss = pl.pallas_call(
        _finalize_kernel,
        out_specs=pl.BlockSpec(memory_space=pltpu.SMEM),
        out_shape=jax.ShapeDtypeStruct((1, 1), jnp.float32),
    )(part_t, t_i32)

    return loss[0, 0]
